# baseline (speedup 1.0000x reference)
import functools

import jax
import jax.numpy as jnp
from jax import lax
from jax.experimental import pallas as pl
from jax.experimental.pallas import tpu as pltpu
from jax.experimental.pallas import tpu_sc as plsc

N = 10000
E = 640000
H = 128
M = 64
LANES = 2 * M

NC = 2
NS = 16
CN = 10
EC = E // CN
EPW = EC // (NC * NS)
CHUNK = 80
NCHUNK = EPW // CHUNK
RQUOT = 624


def _encode_body(x_ref, h_ref, wx_ref, wh_ref, b_ref, p_ref):
    p = jnp.dot(x_ref[...], wx_ref[...], preferred_element_type=jnp.float32)
    p += jnp.dot(h_ref[...], wh_ref[...], preferred_element_type=jnp.float32)
    p_ref[...] = p + b_ref[...]


def _encode(x, h, wx_t, wh_t, b_enc):
    blk = 2000
    return pl.pallas_call(
        _encode_body,
        grid=(N // blk,),
        in_specs=[
            pl.BlockSpec((blk, H), lambda i: (i, 0)),
            pl.BlockSpec((blk, H), lambda i: (i, 0)),
            pl.BlockSpec((H, LANES), lambda i: (0, 0)),
            pl.BlockSpec((H, LANES), lambda i: (0, 0)),
            pl.BlockSpec((1, LANES), lambda i: (0, 0)),
        ],
        out_specs=pl.BlockSpec((blk, LANES), lambda i: (i, 0)),
        out_shape=jax.ShapeDtypeStruct((N, LANES), jnp.float32),
    )(x, h, wx_t, wh_t, b_enc.reshape(1, LANES))


_KS0 = 0
_KS1 = 12345
_KS2 = _KS0 ^ _KS1 ^ 0x1BD11BDA
_ROTS = ((13, 15, 26, 6), (17, 29, 16, 24))


def _i32(v):
    return jnp.int32(((v + 2 ** 31) % 2 ** 32) - 2 ** 31)


def _threefry_bits(cnt):
    sched = (
        (_KS1, _KS2 + 1),
        (_KS2, 0 + 2),
        (0, _KS1 + 3),
        (_KS1, _KS2 + 4),
        (_KS2, 0 + 5),
    )
    x0 = jnp.zeros_like(cnt)
    x1 = cnt + jnp.int32(_KS1)
    for i in range(5):
        for r in _ROTS[i % 2]:
            x0 = x0 + x1
            x1 = lax.shift_left(x1, jnp.int32(r)) | lax.shift_right_logical(
                x1, jnp.int32(32 - r)
            )
            x1 = x1 ^ x0
        a0, a1 = sched[i]
        if a0:
            x0 = x0 + _i32(a0)
        x1 = x1 + _i32(a1)
    return x0 ^ x1


def _gumbel_from_bits(bits):
    tiny = jnp.float32(jnp.finfo(jnp.float32).tiny)
    mant = lax.shift_right_logical(bits, jnp.int32(9)) | jnp.int32(0x3F800000)
    fl = lax.bitcast_convert_type(mant, jnp.float32) - jnp.float32(1.0)
    u = jnp.maximum(tiny, fl)
    return -jnp.log(-jnp.log(u))


def _msg_body(psrc_ref, msg_ref, *, blk, ebase):
    e0 = jnp.int32(ebase) + pl.program_id(0) * blk
    row = lax.broadcasted_iota(jnp.int32, (blk, LANES), 0)
    col = lax.broadcasted_iota(jnp.int32, (blk, LANES), 1)
    cnt = (jnp.int32(e0) + row) * jnp.int32(LANES) + col
    g = _gumbel_from_bits(_threefry_bits(cnt))
    pg = psrc_ref[...] + g
    nxt = jnp.concatenate([pg[:, 1:], pg[:, :1]], axis=1)
    prv = jnp.concatenate([pg[:, -1:], pg[:, :-1]], axis=1)
    even = (col & 1) == 0
    one = jnp.float32(1.0)
    zero = jnp.float32(0.0)
    ge = jnp.where(pg >= nxt, one, zero)
    gt = jnp.where(pg > prv, one, zero)
    msg_ref[...] = jnp.where(even, ge, gt)


def _messages(psrc, ebase):
    blk = 6400
    return pl.pallas_call(
        functools.partial(_msg_body, blk=blk, ebase=ebase),
        grid=(EC // blk,),
        in_specs=[pl.BlockSpec((blk, LANES), lambda i: (i, 0))],
        out_specs=pl.BlockSpec((blk, LANES), lambda i: (i, 0)),
        out_shape=jax.ShapeDtypeStruct((EC, LANES), jnp.float32),
    )(psrc)


def _sc_gather(p, src):
    mesh = plsc.VectorSubcoreMesh(core_axis_name="c", subcore_axis_name="s")

    @functools.partial(
        pl.kernel,
        mesh=mesh,
        out_type=jax.ShapeDtypeStruct((EC, LANES), jnp.float32),
        scratch_types=[
            pltpu.VMEM((EPW,), jnp.int32),
            pltpu.VMEM((CHUNK, LANES), jnp.float32),
            pltpu.VMEM((CHUNK, LANES), jnp.float32),
            pltpu.SemaphoreType.DMA,
            pltpu.SemaphoreType.DMA,
            pltpu.SemaphoreType.DMA,
            pltpu.SemaphoreType.DMA,
        ],
    )
    def k(p_hbm, src_hbm, out_hbm, idx_all, b0, b1, gs0, gs1, ws0, ws1):
        wid = lax.axis_index("s") * NC + lax.axis_index("c")
        base = wid * EPW
        pltpu.sync_copy(src_hbm.at[pl.ds(base, EPW)], idx_all)

        def start_g(i, buf, sem):
            pltpu.async_copy(
                p_hbm.at[idx_all.at[pl.ds(i * CHUNK, CHUNK)]], buf, sem)

        def wait_g(buf, sem):
            pltpu.make_async_copy(
                p_hbm.at[idx_all.at[pl.ds(0, CHUNK)]], buf, sem).wait()

        def start_w(i, buf, sem):
            pltpu.async_copy(
                buf, out_hbm.at[pl.ds(base + i * CHUNK, CHUNK), :], sem)

        def wait_w(buf, sem):
            pltpu.make_async_copy(
                buf, out_hbm.at[pl.ds(base, CHUNK), :], sem).wait()

        start_g(0, b0, gs0)

        def step(i, bufs):
            mine, other, gs_m, gs_o, ws_m, ws_o = bufs

            @pl.when(i + 1 < NCHUNK)
            def _():
                @pl.when(i >= 1)
                def _():
                    wait_w(other, ws_o)
                start_g(i + 1, other, gs_o)

            wait_g(mine, gs_m)
            start_w(i, mine, ws_m)

        def body(i, _):
            @pl.when(i % 2 == 0)
            def _():
                step(i, (b0, b1, gs0, gs1, ws0, ws1))

            @pl.when(i % 2 == 1)
            def _():
                step(i, (b1, b0, gs1, gs0, ws1, ws0))

            return 0

        lax.fori_loop(0, NCHUNK, body, 0)
        wait_w(b0, ws0)
        wait_w(b1, ws1)

    return k(p, src)


def _sc_scatter_add(msg, dst):
    mesh = plsc.VectorSubcoreMesh(core_axis_name="c", subcore_axis_name="s")

    @functools.partial(
        pl.kernel,
        mesh=mesh,
        out_type=jax.ShapeDtypeStruct((NC, N, LANES), jnp.float32),
        scratch_types=[
            pltpu.VMEM((NCHUNK, CHUNK), jnp.int32),
            pltpu.VMEM((CHUNK, LANES), jnp.float32),
            pltpu.VMEM((CHUNK, LANES), jnp.float32),
            pltpu.VMEM((16, LANES), jnp.float32),
            pltpu.VMEM_SHARED((N, LANES), jnp.float32),
            pltpu.SemaphoreType.DMA,
            pltpu.SemaphoreType.DMA,
        ],
    )
    def k(msg_hbm, dst3_hbm, out_hbm, idx_all, m0, m1, zrow_v, acc_sh,
          ls0, ls1):
        cid = lax.axis_index("c")
        sid = lax.axis_index("s")
        wid = cid * NS + sid
        base = wid * EPW
        pltpu.sync_copy(dst3_hbm.at[wid], idx_all)

        def zinit(t, _):
            zrow_v[t // 8, pl.ds((t % 8) * 16, 16)] = jnp.zeros((16,), jnp.float32)
            return 0

        lax.fori_loop(0, 16 * (LANES // 16), zinit, 0)
        r0 = sid * RQUOT
        n16 = jnp.where(sid == NS - 1, (N - (NS - 1) * RQUOT) // 16, RQUOT // 16)

        def zcopy(j, _):
            pltpu.sync_copy(zrow_v, acc_sh.at[pl.ds(r0 + j * 16, 16), :])
            return 0

        lax.fori_loop(0, n16, zcopy, 0)
        plsc.subcore_barrier()

        def start_l(i, buf, sem):
            pltpu.async_copy(
                msg_hbm.at[pl.ds(base + i * CHUNK, CHUNK), :], buf, sem)

        def wait_l(buf, sem):
            pltpu.make_async_copy(
                msg_hbm.at[pl.ds(base, CHUNK), :], buf, sem).wait()

        start_l(0, m0, ls0)

        def step(i, bufs):
            mine, other, ls_m, ls_o = bufs

            @pl.when(i + 1 < NCHUNK)
            def _():
                start_l(i + 1, other, ls_o)

            wait_l(mine, ls_m)
            pltpu.sync_copy(mine, acc_sh.at[idx_all.at[i]], add=True)

        def body(i, _):
            @pl.when(i % 2 == 0)
            def _():
                step(i, (m0, m1, ls0, ls1))

            @pl.when(i % 2 == 1)
            def _():
                step(i, (m1, m0, ls1, ls0))

            return 0

        lax.fori_loop(0, NCHUNK, body, 0)
        plsc.subcore_barrier()

        @pl.when(sid < NS - 1)
        def _():
            pltpu.sync_copy(acc_sh.at[pl.ds(r0, RQUOT), :],
                            out_hbm.at[cid, pl.ds(r0, RQUOT), :])

        @pl.when(sid == NS - 1)
        def _():
            rt = (NS - 1) * RQUOT
            pltpu.sync_copy(acc_sh.at[pl.ds(rt, N - rt), :],
                            out_hbm.at[cid, pl.ds(rt, N - rt), :])

    return k(msg, dst)


def _gru_body(x_ref, h_ref, *rest):
    cs_refs = rest[:2 * CN]
    (wd_ref, bd_ref, wix_ref, wic_ref, bi_ref, wh_ref, bh_ref,
     out_ref) = rest[2 * CN:]
    one = jnp.float32(1.0)
    zero = jnp.float32(0.0)
    cs = cs_refs[0][...]
    for r in cs_refs[1:]:
        cs = cs + r[...]
    c = jnp.where(cs > 0.0, one, zero)
    dec = jnp.dot(c, wd_ref[...], preferred_element_type=jnp.float32)
    dec += bd_ref[...]
    gi = jnp.dot(x_ref[...], wix_ref[...], preferred_element_type=jnp.float32)
    gi += jnp.dot(dec, wic_ref[...], preferred_element_type=jnp.float32)
    gi += bi_ref[...]
    gh = jnp.dot(h_ref[...], wh_ref[...], preferred_element_type=jnp.float32)
    gh += bh_ref[...]
    i_r, i_z, i_n = gi[:, :H], gi[:, H:2 * H], gi[:, 2 * H:]
    h_r, h_z, h_n = gh[:, :H], gh[:, H:2 * H], gh[:, 2 * H:]
    r = jax.nn.sigmoid(i_r + h_r)
    z = jax.nn.sigmoid(i_z + h_z)
    n = jnp.tanh(i_n + r * h_n)
    out_ref[...] = (1.0 - z) * n + z * h_ref[...]


def _gru(x, h, cs_list, wd_t, b_dec, wix_t, wic_t, b_ih, whh_t, b_hh):
    blk = 2000
    return pl.pallas_call(
        _gru_body,
        grid=(N // blk,),
        in_specs=[
            pl.BlockSpec((blk, H), lambda i: (i, 0)),
            pl.BlockSpec((blk, H), lambda i: (i, 0)),
        ] + [
            pl.BlockSpec((blk, LANES), lambda i: (i, 0))
            for _ in range(2 * CN)
        ] + [
            pl.BlockSpec((LANES, LANES), lambda i: (0, 0)),
            pl.BlockSpec((1, LANES), lambda i: (0, 0)),
            pl.BlockSpec((H, 3 * H), lambda i: (0, 0)),
            pl.BlockSpec((LANES, 3 * H), lambda i: (0, 0)),
            pl.BlockSpec((1, 3 * H), lambda i: (0, 0)),
            pl.BlockSpec((H, 3 * H), lambda i: (0, 0)),
            pl.BlockSpec((1, 3 * H), lambda i: (0, 0)),
        ],
        out_specs=pl.BlockSpec((blk, H), lambda i: (i, 0)),
        out_shape=jax.ShapeDtypeStruct((N, H), jnp.float32),
    )(x, h, *cs_list, wd_t, b_dec.reshape(1, LANES), wix_t, wic_t,
      b_ih.reshape(1, 3 * H), whh_t, b_hh.reshape(1, 3 * H))


def kernel(x, h, edge_index, W_enc, b_enc, W_dec, b_dec, W_ih, b_ih, W_hh, b_hh):
    src = edge_index[0]
    dst = edge_index[1]
    wx_t = W_enc[:, :H].T
    wh_t = W_enc[:, H:].T
    p = _encode(x, h, wx_t, wh_t, b_enc)
    cs_list = []
    for i in range(CN):
        sl = slice(i * EC, (i + 1) * EC)
        psrc = _sc_gather(p, src[sl])
        msg = _messages(psrc, i * EC)
        dst3 = dst[sl].reshape(NC * NS, NCHUNK, CHUNK)
        csum = _sc_scatter_add(msg, dst3)
        cs_list.append(csum[0])
        cs_list.append(csum[1])
    return _gru(x, h, cs_list, W_dec.T, b_dec, W_ih[:, :H].T,
                W_ih[:, H:].T, b_ih, W_hh.T, b_hh)


# speedup vs baseline: 1.0011x; 1.0011x over previous
import functools

import jax
import jax.numpy as jnp
from jax import lax
from jax.experimental import pallas as pl
from jax.experimental.pallas import tpu as pltpu
from jax.experimental.pallas import tpu_sc as plsc

N = 10000
E = 640000
H = 128
M = 64
LANES = 2 * M

NC = 2
NS = 16
CN = 10
EC = E // CN
EPW = EC // (NC * NS)
CHUNK = 80
NCHUNK = EPW // CHUNK
RQUOT = 624


def _encode_body(x_ref, h_ref, wx_ref, wh_ref, b_ref, p_ref):
    p = jnp.dot(x_ref[...], wx_ref[...], preferred_element_type=jnp.float32)
    p += jnp.dot(h_ref[...], wh_ref[...], preferred_element_type=jnp.float32)
    p_ref[...] = p + b_ref[...]


def _encode(x, h, wx_t, wh_t, b_enc):
    blk = 2000
    return pl.pallas_call(
        _encode_body,
        grid=(N // blk,),
        in_specs=[
            pl.BlockSpec((blk, H), lambda i: (i, 0)),
            pl.BlockSpec((blk, H), lambda i: (i, 0)),
            pl.BlockSpec((H, LANES), lambda i: (0, 0)),
            pl.BlockSpec((H, LANES), lambda i: (0, 0)),
            pl.BlockSpec((1, LANES), lambda i: (0, 0)),
        ],
        out_specs=pl.BlockSpec((blk, LANES), lambda i: (i, 0)),
        out_shape=jax.ShapeDtypeStruct((N, LANES), jnp.float32),
    )(x, h, wx_t, wh_t, b_enc.reshape(1, LANES))


_KS0 = 0
_KS1 = 12345
_KS2 = _KS0 ^ _KS1 ^ 0x1BD11BDA
_ROTS = ((13, 15, 26, 6), (17, 29, 16, 24))


def _i32(v):
    return jnp.int32(((v + 2 ** 31) % 2 ** 32) - 2 ** 31)


def _threefry_bits(cnt):
    sched = (
        (_KS1, _KS2 + 1),
        (_KS2, 0 + 2),
        (0, _KS1 + 3),
        (_KS1, _KS2 + 4),
        (_KS2, 0 + 5),
    )
    x0 = jnp.zeros_like(cnt)
    x1 = cnt + jnp.int32(_KS1)
    for i in range(5):
        for r in _ROTS[i % 2]:
            x0 = x0 + x1
            x1 = lax.shift_left(x1, jnp.int32(r)) | lax.shift_right_logical(
                x1, jnp.int32(32 - r)
            )
            x1 = x1 ^ x0
        a0, a1 = sched[i]
        if a0:
            x0 = x0 + _i32(a0)
        x1 = x1 + _i32(a1)
    return x0 ^ x1


def _gumbel_from_bits(bits):
    tiny = jnp.float32(jnp.finfo(jnp.float32).tiny)
    mant = lax.shift_right_logical(bits, jnp.int32(9)) | jnp.int32(0x3F800000)
    fl = lax.bitcast_convert_type(mant, jnp.float32) - jnp.float32(1.0)
    u = jnp.maximum(tiny, fl)
    return -jnp.log(-jnp.log(u))


def _msg_body(psrc_ref, msg_ref, *, blk, ebase):
    e0 = jnp.int32(ebase) + pl.program_id(0) * blk
    row = lax.broadcasted_iota(jnp.int32, (blk, LANES), 0)
    col = lax.broadcasted_iota(jnp.int32, (blk, LANES), 1)
    cnt = (jnp.int32(e0) + row) * jnp.int32(LANES) + col
    g = _gumbel_from_bits(_threefry_bits(cnt))
    pg = psrc_ref[...] + g
    nxt = jnp.concatenate([pg[:, 1:], pg[:, :1]], axis=1)
    prv = jnp.concatenate([pg[:, -1:], pg[:, :-1]], axis=1)
    even = (col & 1) == 0
    one = jnp.float32(1.0)
    zero = jnp.float32(0.0)
    ge = jnp.where(pg >= nxt, one, zero)
    gt = jnp.where(pg > prv, one, zero)
    msg_ref[...] = jnp.where(even, ge, gt)


def _messages(psrc, ebase):
    blk = 3200
    return pl.pallas_call(
        functools.partial(_msg_body, blk=blk, ebase=ebase),
        grid=(EC // blk,),
        in_specs=[pl.BlockSpec((blk, LANES), lambda i: (i, 0))],
        out_specs=pl.BlockSpec((blk, LANES), lambda i: (i, 0)),
        out_shape=jax.ShapeDtypeStruct((EC, LANES), jnp.float32),
    )(psrc)


def _sc_gather(p, src):
    mesh = plsc.VectorSubcoreMesh(core_axis_name="c", subcore_axis_name="s")

    @functools.partial(
        pl.kernel,
        mesh=mesh,
        out_type=jax.ShapeDtypeStruct((EC, LANES), jnp.float32),
        scratch_types=[
            pltpu.VMEM((EPW,), jnp.int32),
            pltpu.VMEM((CHUNK, LANES), jnp.float32),
            pltpu.VMEM((CHUNK, LANES), jnp.float32),
            pltpu.SemaphoreType.DMA,
            pltpu.SemaphoreType.DMA,
            pltpu.SemaphoreType.DMA,
            pltpu.SemaphoreType.DMA,
        ],
    )
    def k(p_hbm, src_hbm, out_hbm, idx_all, b0, b1, gs0, gs1, ws0, ws1):
        wid = lax.axis_index("s") * NC + lax.axis_index("c")
        base = wid * EPW
        pltpu.sync_copy(src_hbm.at[pl.ds(base, EPW)], idx_all)

        def start_g(i, buf, sem):
            pltpu.async_copy(
                p_hbm.at[idx_all.at[pl.ds(i * CHUNK, CHUNK)]], buf, sem)

        def wait_g(buf, sem):
            pltpu.make_async_copy(
                p_hbm.at[idx_all.at[pl.ds(0, CHUNK)]], buf, sem).wait()

        def start_w(i, buf, sem):
            pltpu.async_copy(
                buf, out_hbm.at[pl.ds(base + i * CHUNK, CHUNK), :], sem)

        def wait_w(buf, sem):
            pltpu.make_async_copy(
                buf, out_hbm.at[pl.ds(base, CHUNK), :], sem).wait()

        start_g(0, b0, gs0)

        def step(i, bufs):
            mine, other, gs_m, gs_o, ws_m, ws_o = bufs

            @pl.when(i + 1 < NCHUNK)
            def _():
                @pl.when(i >= 1)
                def _():
                    wait_w(other, ws_o)
                start_g(i + 1, other, gs_o)

            wait_g(mine, gs_m)
            start_w(i, mine, ws_m)

        def body(i, _):
            @pl.when(i % 2 == 0)
            def _():
                step(i, (b0, b1, gs0, gs1, ws0, ws1))

            @pl.when(i % 2 == 1)
            def _():
                step(i, (b1, b0, gs1, gs0, ws1, ws0))

            return 0

        lax.fori_loop(0, NCHUNK, body, 0)
        wait_w(b0, ws0)
        wait_w(b1, ws1)

    return k(p, src)


def _sc_scatter_add(msg, dst):
    mesh = plsc.VectorSubcoreMesh(core_axis_name="c", subcore_axis_name="s")

    @functools.partial(
        pl.kernel,
        mesh=mesh,
        out_type=jax.ShapeDtypeStruct((NC, N, LANES), jnp.float32),
        scratch_types=[
            pltpu.VMEM((NCHUNK, CHUNK), jnp.int32),
            pltpu.VMEM((CHUNK, LANES), jnp.float32),
            pltpu.VMEM((CHUNK, LANES), jnp.float32),
            pltpu.VMEM((16, LANES), jnp.float32),
            pltpu.VMEM_SHARED((N, LANES), jnp.float32),
            pltpu.SemaphoreType.DMA,
            pltpu.SemaphoreType.DMA,
        ],
    )
    def k(msg_hbm, dst3_hbm, out_hbm, idx_all, m0, m1, zrow_v, acc_sh,
          ls0, ls1):
        cid = lax.axis_index("c")
        sid = lax.axis_index("s")
        wid = cid * NS + sid
        base = wid * EPW
        pltpu.sync_copy(dst3_hbm.at[wid], idx_all)

        def zinit(t, _):
            zrow_v[t // 8, pl.ds((t % 8) * 16, 16)] = jnp.zeros((16,), jnp.float32)
            return 0

        lax.fori_loop(0, 16 * (LANES // 16), zinit, 0)
        r0 = sid * RQUOT
        n16 = jnp.where(sid == NS - 1, (N - (NS - 1) * RQUOT) // 16, RQUOT // 16)

        def zcopy(j, _):
            pltpu.sync_copy(zrow_v, acc_sh.at[pl.ds(r0 + j * 16, 16), :])
            return 0

        lax.fori_loop(0, n16, zcopy, 0)
        plsc.subcore_barrier()

        def start_l(i, buf, sem):
            pltpu.async_copy(
                msg_hbm.at[pl.ds(base + i * CHUNK, CHUNK), :], buf, sem)

        def wait_l(buf, sem):
            pltpu.make_async_copy(
                msg_hbm.at[pl.ds(base, CHUNK), :], buf, sem).wait()

        start_l(0, m0, ls0)

        def step(i, bufs):
            mine, other, ls_m, ls_o = bufs

            @pl.when(i + 1 < NCHUNK)
            def _():
                start_l(i + 1, other, ls_o)

            wait_l(mine, ls_m)
            pltpu.sync_copy(mine, acc_sh.at[idx_all.at[i]], add=True)

        def body(i, _):
            @pl.when(i % 2 == 0)
            def _():
                step(i, (m0, m1, ls0, ls1))

            @pl.when(i % 2 == 1)
            def _():
                step(i, (m1, m0, ls1, ls0))

            return 0

        lax.fori_loop(0, NCHUNK, body, 0)
        plsc.subcore_barrier()

        @pl.when(sid < NS - 1)
        def _():
            pltpu.sync_copy(acc_sh.at[pl.ds(r0, RQUOT), :],
                            out_hbm.at[cid, pl.ds(r0, RQUOT), :])

        @pl.when(sid == NS - 1)
        def _():
            rt = (NS - 1) * RQUOT
            pltpu.sync_copy(acc_sh.at[pl.ds(rt, N - rt), :],
                            out_hbm.at[cid, pl.ds(rt, N - rt), :])

    return k(msg, dst)


def _gru_body(x_ref, h_ref, *rest):
    cs_refs = rest[:2 * CN]
    (wd_ref, bd_ref, wix_ref, wic_ref, bi_ref, wh_ref, bh_ref,
     out_ref) = rest[2 * CN:]
    one = jnp.float32(1.0)
    zero = jnp.float32(0.0)
    cs = cs_refs[0][...]
    for r in cs_refs[1:]:
        cs = cs + r[...]
    c = jnp.where(cs > 0.0, one, zero)
    dec = jnp.dot(c, wd_ref[...], preferred_element_type=jnp.float32)
    dec += bd_ref[...]
    gi = jnp.dot(x_ref[...], wix_ref[...], preferred_element_type=jnp.float32)
    gi += jnp.dot(dec, wic_ref[...], preferred_element_type=jnp.float32)
    gi += bi_ref[...]
    gh = jnp.dot(h_ref[...], wh_ref[...], preferred_element_type=jnp.float32)
    gh += bh_ref[...]
    i_r, i_z, i_n = gi[:, :H], gi[:, H:2 * H], gi[:, 2 * H:]
    h_r, h_z, h_n = gh[:, :H], gh[:, H:2 * H], gh[:, 2 * H:]
    r = jax.nn.sigmoid(i_r + h_r)
    z = jax.nn.sigmoid(i_z + h_z)
    n = jnp.tanh(i_n + r * h_n)
    out_ref[...] = (1.0 - z) * n + z * h_ref[...]


def _gru(x, h, cs_list, wd_t, b_dec, wix_t, wic_t, b_ih, whh_t, b_hh):
    blk = 2000
    return pl.pallas_call(
        _gru_body,
        grid=(N // blk,),
        in_specs=[
            pl.BlockSpec((blk, H), lambda i: (i, 0)),
            pl.BlockSpec((blk, H), lambda i: (i, 0)),
        ] + [
            pl.BlockSpec((blk, LANES), lambda i: (i, 0))
            for _ in range(2 * CN)
        ] + [
            pl.BlockSpec((LANES, LANES), lambda i: (0, 0)),
            pl.BlockSpec((1, LANES), lambda i: (0, 0)),
            pl.BlockSpec((H, 3 * H), lambda i: (0, 0)),
            pl.BlockSpec((LANES, 3 * H), lambda i: (0, 0)),
            pl.BlockSpec((1, 3 * H), lambda i: (0, 0)),
            pl.BlockSpec((H, 3 * H), lambda i: (0, 0)),
            pl.BlockSpec((1, 3 * H), lambda i: (0, 0)),
        ],
        out_specs=pl.BlockSpec((blk, H), lambda i: (i, 0)),
        out_shape=jax.ShapeDtypeStruct((N, H), jnp.float32),
    )(x, h, *cs_list, wd_t, b_dec.reshape(1, LANES), wix_t, wic_t,
      b_ih.reshape(1, 3 * H), whh_t, b_hh.reshape(1, 3 * H))


def kernel(x, h, edge_index, W_enc, b_enc, W_dec, b_dec, W_ih, b_ih, W_hh, b_hh):
    src = edge_index[0]
    dst = edge_index[1]
    wx_t = W_enc[:, :H].T
    wh_t = W_enc[:, H:].T
    p = _encode(x, h, wx_t, wh_t, b_enc)
    cs_list = []
    for i in range(CN):
        sl = slice(i * EC, (i + 1) * EC)
        psrc = _sc_gather(p, src[sl])
        msg = _messages(psrc, i * EC)
        dst3 = dst[sl].reshape(NC * NS, NCHUNK, CHUNK)
        csum = _sc_scatter_add(msg, dst3)
        cs_list.append(csum[0])
        cs_list.append(csum[1])
    return _gru(x, h, cs_list, W_dec.T, b_dec, W_ih[:, :H].T,
                W_ih[:, H:].T, b_ih, W_hh.T, b_hh)
